# baseline (speedup 1.0000x reference)
import math

import jax
import jax.numpy as jnp
from jax import lax
from jax.experimental import pallas as pl
from jax.experimental.pallas import tpu as pltpu
from jax.experimental.pallas import tpu_sc as plsc

N_ATOMS = 10000
N_EDGES = 320000
EMB = 128
EMB_RBF = 16

NC = 2
NS = 16
NW = NC * NS

CHUNK = 80
BLOCK1 = 12800
NBLK_A = 12
NBLK_B = 13
E_PART_A = NBLK_A * BLOCK1
CHUNKS_A = E_PART_A // NW // CHUNK
CHUNKS_B = (NBLK_B * BLOCK1) // NW // CHUNK
A_PAD = 10240
A_PER_S = A_PAD // NS
ZROWS = 40

_INV_SQRT2 = 1.0 / math.sqrt(2.0)


def _scaled_silu(x):
    return x * jax.lax.logistic(x) * (1.0 / 0.6)


def _edge_msg_body(basis_ref, m_ref, w_ref, out_ref):
    bases = jnp.dot(basis_ref[...], w_ref[...], preferred_element_type=jnp.float32)
    x = m_ref[...] * bases
    bits = lax.bitcast_convert_type(x.astype(jnp.bfloat16), jnp.uint16)
    bits = bits.astype(jnp.uint32)
    nb = bits.shape[0]
    g = bits.reshape(nb // CHUNK, CHUNK, EMB)
    lo = g[:, : CHUNK // 2, :]
    hi = g[:, CHUNK // 2 :, :]
    out_ref[...] = (lo | (hi << 16)).reshape(nb // 2, EMB)


def _edge_messages(basis_rad, m, W_rbf, off_blocks, n_blocks):
    return pl.pallas_call(
        _edge_msg_body,
        grid=(n_blocks,),
        in_specs=[
            pl.BlockSpec((BLOCK1, EMB_RBF), lambda i: (i + off_blocks, 0)),
            pl.BlockSpec((BLOCK1, EMB), lambda i: (i + off_blocks, 0)),
            pl.BlockSpec((EMB_RBF, EMB), lambda i: (0, 0)),
        ],
        out_specs=pl.BlockSpec((BLOCK1 // 2, EMB), lambda i: (i, 0)),
        out_shape=jax.ShapeDtypeStruct((n_blocks * BLOCK1 // 2, EMB), jnp.uint32),
    )(basis_rad, m, W_rbf)


def _make_scatter_body(e_base, n_chunks):
    e_per_w = n_chunks * CHUNK

    def _scatter_body(x_hbm, idx_hbm, out_hbm, idx0, idx1, idxsc0, idxsc1,
                      raw0, raw1, rows0, rows1,
                      zero_v, acc_sh, si0, si1, sr0, sr1, ss0, ss1):
        cid = lax.axis_index("c")
        sid = lax.axis_index("s")
        wid = sid * NC + cid
        base_e = e_base + wid * e_per_w
        base_h = wid * (e_per_w // 2)

        idxs = (idx0, idx1)
        idxscs = (idxsc0, idxsc1)
        raws = (raw0, raw1)
        rows = (rows0, rows1)
        sis = (si0, si1)
        srs = (sr0, sr1)
        sss = (ss0, ss1)

        def _fill_zero(i, _):
            r = i // 8
            c = lax.rem(i, 8)
            zero_v[r, pl.ds(c * 16, 16)] = jnp.zeros((16,), jnp.float32)
            return 0

        lax.fori_loop(0, ZROWS * 8, _fill_zero, 0)

        def _zero_acc(j, _):
            pltpu.sync_copy(zero_v,
                            acc_sh.at[pl.ds(sid * A_PER_S + j * ZROWS, ZROWS)])
            return 0

        lax.fori_loop(0, A_PER_S // ZROWS, _zero_acc, 0)
        plsc.subcore_barrier()

        def _start_gather(g, b):
            e0 = base_e + g * CHUNK
            h0 = base_h + g * (CHUNK // 2)
            pltpu.async_copy(idx_hbm.at[pl.ds(e0, CHUNK)], idxs[b], sis[b])
            pltpu.async_copy(x_hbm.at[pl.ds(h0, CHUNK // 2)], raws[b], srs[b])

        def _wait_gather(g, b):
            e0 = base_e + g * CHUNK
            h0 = base_h + g * (CHUNK // 2)
            pltpu.make_async_copy(idx_hbm.at[pl.ds(e0, CHUNK)], idxs[b],
                                  sis[b]).wait()
            pltpu.make_async_copy(x_hbm.at[pl.ds(h0, CHUNK // 2)], raws[b],
                                  srs[b]).wait()

        def _unpack(b):
            raw = raws[b]
            dst = rows[b]

            @plsc.parallel_loop(0, CHUNK // 2, unroll=4)
            def _rowpair(r2):
                for g8 in range(EMB // 16):
                    w = raw[r2, pl.ds(g8 * 16, 16)]
                    bb = plsc.bitcast(w, jnp.bfloat16)
                    lo, hi = plsc.unpack(bb, format=plsc.PackFormat.INTERLEAVED)
                    dst[r2, pl.ds(g8 * 16, 16)] = lo
                    dst[r2 + CHUNK // 2, pl.ds(g8 * 16, 16)] = hi

        def _copy_idx(b):
            for j in range(CHUNK // 16):
                idxscs[b][pl.ds(j * 16, 16)] = idxs[b][pl.ds(j * 16, 16)]

        def _start_scatter(b):
            pltpu.async_copy(rows[b], acc_sh.at[idxscs[b]], sss[b], add=True)

        def _wait_scatter(b):
            pltpu.make_async_copy(rows[b], acc_sh.at[idxscs[b]], sss[b]).wait()

        _start_gather(0, 0)
        _start_gather(1, 1)
        _wait_gather(0, 0)
        _unpack(0)
        _copy_idx(0)
        _start_gather(2, 0)
        _start_scatter(0)
        _wait_gather(1, 1)
        _unpack(1)
        _copy_idx(1)
        _start_gather(3, 1)
        _start_scatter(1)

        def _pair(p, _):
            ga = 2 * p
            gb = 2 * p + 1
            _wait_gather(ga, 0)
            _wait_scatter(0)
            _unpack(0)
            _copy_idx(0)
            _start_gather(ga + 2, 0)
            _start_scatter(0)
            _wait_gather(gb, 1)
            _wait_scatter(1)
            _unpack(1)
            _copy_idx(1)
            _start_gather(gb + 2, 1)
            _start_scatter(1)
            return 0

        if n_chunks % 2:
            lax.fori_loop(1, (n_chunks - 3) // 2, _pair, 0)
            _wait_gather(n_chunks - 3, 0)
            _wait_scatter(0)
            _unpack(0)
            _copy_idx(0)
            _start_gather(n_chunks - 1, 0)
            _start_scatter(0)
            _wait_gather(n_chunks - 2, 1)
            _wait_scatter(1)
            _unpack(1)
            _copy_idx(1)
            _start_scatter(1)
            _wait_gather(n_chunks - 1, 0)
            _wait_scatter(0)
            _unpack(0)
            _copy_idx(0)
            _start_scatter(0)
            _wait_scatter(1)
            _wait_scatter(0)
        else:
            lax.fori_loop(1, (n_chunks - 2) // 2, _pair, 0)
            _wait_gather(n_chunks - 2, 0)
            _wait_scatter(0)
            _unpack(0)
            _copy_idx(0)
            _start_scatter(0)
            _wait_gather(n_chunks - 1, 1)
            _wait_scatter(1)
            _unpack(1)
            _copy_idx(1)
            _start_scatter(1)
            _wait_scatter(0)
            _wait_scatter(1)
        plsc.subcore_barrier()

        pltpu.sync_copy(
            acc_sh.at[pl.ds(sid * A_PER_S, A_PER_S)],
            out_hbm.at[cid, pl.ds(sid * A_PER_S, A_PER_S)],
        )

    return _scatter_body


def _scatter_partials(x, idx_atom, e_base, n_chunks):
    mesh = plsc.VectorSubcoreMesh(core_axis_name="c", subcore_axis_name="s")
    k = pl.kernel(
        _make_scatter_body(e_base, n_chunks),
        out_type=jax.ShapeDtypeStruct((NC, A_PAD, EMB), jnp.float32),
        mesh=mesh,
        scratch_types=[
            pltpu.VMEM((CHUNK,), jnp.int32),
            pltpu.VMEM((CHUNK,), jnp.int32),
            pltpu.VMEM((CHUNK,), jnp.int32),
            pltpu.VMEM((CHUNK,), jnp.int32),
            pltpu.VMEM((CHUNK // 2, EMB), jnp.uint32),
            pltpu.VMEM((CHUNK // 2, EMB), jnp.uint32),
            pltpu.VMEM((CHUNK, EMB), jnp.float32),
            pltpu.VMEM((CHUNK, EMB), jnp.float32),
            pltpu.VMEM((ZROWS, EMB), jnp.float32),
            pltpu.VMEM_SHARED((A_PAD, EMB), jnp.float32),
            pltpu.SemaphoreType.DMA,
            pltpu.SemaphoreType.DMA,
            pltpu.SemaphoreType.DMA,
            pltpu.SemaphoreType.DMA,
            pltpu.SemaphoreType.DMA,
            pltpu.SemaphoreType.DMA,
        ],
        compiler_params=pltpu.CompilerParams(needs_layout_passes=False),
    )
    return k(x, idx_atom)


def _mlp_body(scale_ref, pa_ref, pb_ref, w10_ref, w20_ref, w11_ref, w21_ref,
              out_ref):
    x = (pa_ref[0] + pa_ref[1] + pb_ref[0] + pb_ref[1]) * scale_ref[0]
    for wa_ref, wb_ref in ((w10_ref, w20_ref), (w11_ref, w21_ref)):
        y = _scaled_silu(jnp.dot(x, wa_ref[...], preferred_element_type=jnp.float32))
        y = _scaled_silu(jnp.dot(y, wb_ref[...], preferred_element_type=jnp.float32))
        x = (x + y) * _INV_SQRT2
    out_ref[...] = x


def _residual_mlp(pa, pb, scale, W1_0, W2_0, W1_1, W2_1, block=2000):
    grid = (N_ATOMS // block,)
    wspec = pl.BlockSpec((EMB, EMB), lambda i: (0, 0))
    pspec = pl.BlockSpec((NC, block, EMB), lambda i: (0, i, 0))
    return pl.pallas_call(
        _mlp_body,
        grid=grid,
        in_specs=[
            pl.BlockSpec(memory_space=pltpu.SMEM),
            pspec, pspec,
            wspec, wspec, wspec, wspec,
        ],
        out_specs=pl.BlockSpec((block, EMB), lambda i: (i, 0)),
        out_shape=jax.ShapeDtypeStruct((N_ATOMS, EMB), jnp.float32),
    )(scale.reshape((1,)), pa, pb, W1_0, W2_0, W1_1, W2_1)


def kernel(h, m, basis_rad, idx_atom, W_rbf, W1_0, W2_0, W1_1, W2_1, scale):
    idx32 = idx_atom.astype(jnp.int32)
    xa = _edge_messages(basis_rad, m, W_rbf, 0, NBLK_A)
    pa = _scatter_partials(xa, idx32, 0, CHUNKS_A)
    xb = _edge_messages(basis_rad, m, W_rbf, NBLK_A, NBLK_B)
    pb = _scatter_partials(xb, idx32, E_PART_A, CHUNKS_B)
    return _residual_mlp(pa, pb, scale, W1_0, W2_0, W1_1, W2_1)


# speedup vs baseline: 1.0141x; 1.0141x over previous
import math

import jax
import jax.numpy as jnp
from jax import lax
from jax.experimental import pallas as pl
from jax.experimental.pallas import tpu as pltpu
from jax.experimental.pallas import tpu_sc as plsc

N_ATOMS = 10000
N_EDGES = 320000
EMB = 128
EMB_RBF = 16

NC = 2
NS = 16
NW = NC * NS

CHUNK = 80
BLOCK1 = 12800
NBLK_A = 12
NBLK_B = 13
E_PART_A = NBLK_A * BLOCK1
CHUNKS_A = E_PART_A // NW // CHUNK
CHUNKS_B = (NBLK_B * BLOCK1) // NW // CHUNK
A_PAD = 10240
A_PER_S = A_PAD // NS
ZROWS = 40

_INV_SQRT2 = 1.0 / math.sqrt(2.0)


def _scaled_silu(x):
    return x * jax.lax.logistic(x) * (1.0 / 0.6)


def _edge_msg_body(basis_ref, m_ref, w_ref, out_ref):
    bases = jnp.dot(basis_ref[...], w_ref[...], preferred_element_type=jnp.float32)
    x = m_ref[...] * bases
    bits = lax.bitcast_convert_type(x.astype(jnp.bfloat16), jnp.uint16)
    bits = bits.astype(jnp.uint32)
    nb = bits.shape[0]
    g = bits.reshape(nb // CHUNK, CHUNK, EMB)
    lo = g[:, : CHUNK // 2, :]
    hi = g[:, CHUNK // 2 :, :]
    out_ref[...] = (lo | (hi << 16)).reshape(nb // 2, EMB)


def _edge_messages(basis_rad, m, W_rbf, off_blocks, n_blocks):
    return pl.pallas_call(
        _edge_msg_body,
        grid=(n_blocks,),
        in_specs=[
            pl.BlockSpec((BLOCK1, EMB_RBF), lambda i: (i + off_blocks, 0)),
            pl.BlockSpec((BLOCK1, EMB), lambda i: (i + off_blocks, 0)),
            pl.BlockSpec((EMB_RBF, EMB), lambda i: (0, 0)),
        ],
        out_specs=pl.BlockSpec((BLOCK1 // 2, EMB), lambda i: (i, 0)),
        out_shape=jax.ShapeDtypeStruct((n_blocks * BLOCK1 // 2, EMB), jnp.uint32),
    )(basis_rad, m, W_rbf)


def _make_scatter_body(e_base, n_chunks):
    e_per_w = n_chunks * CHUNK

    def _scatter_body(x_hbm, idx_hbm, out_hbm, idx0, idx1, idxsc0, idxsc1,
                      raw0, raw1, rows0, rows1,
                      zero_v, acc_sh, si0, si1, sr0, sr1, ss0, ss1):
        cid = lax.axis_index("c")
        sid = lax.axis_index("s")
        wid = sid * NC + cid
        base_e = e_base + wid * e_per_w
        base_h = wid * (e_per_w // 2)

        idxs = (idx0, idx1)
        idxscs = (idxsc0, idxsc1)
        raws = (raw0, raw1)
        rows = (rows0, rows1)
        sis = (si0, si1)
        srs = (sr0, sr1)
        sss = (ss0, ss1)

        def _fill_zero(i, _):
            r = i // 8
            c = lax.rem(i, 8)
            zero_v[r, pl.ds(c * 16, 16)] = jnp.zeros((16,), jnp.float32)
            return 0

        lax.fori_loop(0, ZROWS * 8, _fill_zero, 0)

        def _zero_acc(j, _):
            pltpu.sync_copy(zero_v,
                            acc_sh.at[pl.ds(sid * A_PER_S + j * ZROWS, ZROWS)])
            return 0

        lax.fori_loop(0, A_PER_S // ZROWS, _zero_acc, 0)
        plsc.subcore_barrier()

        def _start_gather(g, b):
            e0 = base_e + g * CHUNK
            h0 = base_h + g * (CHUNK // 2)
            pltpu.async_copy(idx_hbm.at[pl.ds(e0, CHUNK)], idxs[b], sis[b])
            pltpu.async_copy(x_hbm.at[pl.ds(h0, CHUNK // 2)], raws[b], srs[b])

        def _wait_gather(g, b):
            e0 = base_e + g * CHUNK
            h0 = base_h + g * (CHUNK // 2)
            pltpu.make_async_copy(idx_hbm.at[pl.ds(e0, CHUNK)], idxs[b],
                                  sis[b]).wait()
            pltpu.make_async_copy(x_hbm.at[pl.ds(h0, CHUNK // 2)], raws[b],
                                  srs[b]).wait()

        def _unpack(b):
            raw = raws[b]
            dst = rows[b]

            def _rowpair(r2, _):
                for g8 in range(EMB // 16):
                    w = raw[r2, pl.ds(g8 * 16, 16)]
                    bb = plsc.bitcast(w, jnp.bfloat16)
                    lo, hi = plsc.unpack(bb, format=plsc.PackFormat.INTERLEAVED)
                    dst[r2, pl.ds(g8 * 16, 16)] = lo
                    dst[r2 + CHUNK // 2, pl.ds(g8 * 16, 16)] = hi
                return 0

            lax.fori_loop(0, CHUNK // 2, _rowpair, 0)

        def _copy_idx(b):
            for j in range(CHUNK // 16):
                idxscs[b][pl.ds(j * 16, 16)] = idxs[b][pl.ds(j * 16, 16)]

        def _start_scatter(b):
            pltpu.async_copy(rows[b], acc_sh.at[idxscs[b]], sss[b], add=True)

        def _wait_scatter(b):
            pltpu.make_async_copy(rows[b], acc_sh.at[idxscs[b]], sss[b]).wait()

        _start_gather(0, 0)
        _start_gather(1, 1)
        _wait_gather(0, 0)
        _unpack(0)
        _copy_idx(0)
        _start_gather(2, 0)
        _start_scatter(0)
        _wait_gather(1, 1)
        _unpack(1)
        _copy_idx(1)
        _start_gather(3, 1)
        _start_scatter(1)

        def _pair(p, _):
            ga = 2 * p
            gb = 2 * p + 1
            _wait_gather(ga, 0)
            _wait_scatter(0)
            _unpack(0)
            _copy_idx(0)
            _start_gather(ga + 2, 0)
            _start_scatter(0)
            _wait_gather(gb, 1)
            _wait_scatter(1)
            _unpack(1)
            _copy_idx(1)
            _start_gather(gb + 2, 1)
            _start_scatter(1)
            return 0

        if n_chunks % 2:
            lax.fori_loop(1, (n_chunks - 3) // 2, _pair, 0)
            _wait_gather(n_chunks - 3, 0)
            _wait_scatter(0)
            _unpack(0)
            _copy_idx(0)
            _start_gather(n_chunks - 1, 0)
            _start_scatter(0)
            _wait_gather(n_chunks - 2, 1)
            _wait_scatter(1)
            _unpack(1)
            _copy_idx(1)
            _start_scatter(1)
            _wait_gather(n_chunks - 1, 0)
            _wait_scatter(0)
            _unpack(0)
            _copy_idx(0)
            _start_scatter(0)
            _wait_scatter(1)
            _wait_scatter(0)
        else:
            lax.fori_loop(1, (n_chunks - 2) // 2, _pair, 0)
            _wait_gather(n_chunks - 2, 0)
            _wait_scatter(0)
            _unpack(0)
            _copy_idx(0)
            _start_scatter(0)
            _wait_gather(n_chunks - 1, 1)
            _wait_scatter(1)
            _unpack(1)
            _copy_idx(1)
            _start_scatter(1)
            _wait_scatter(0)
            _wait_scatter(1)
        plsc.subcore_barrier()

        pltpu.sync_copy(
            acc_sh.at[pl.ds(sid * A_PER_S, A_PER_S)],
            out_hbm.at[cid, pl.ds(sid * A_PER_S, A_PER_S)],
        )

    return _scatter_body


def _scatter_partials(x, idx_atom, e_base, n_chunks):
    mesh = plsc.VectorSubcoreMesh(core_axis_name="c", subcore_axis_name="s")
    k = pl.kernel(
        _make_scatter_body(e_base, n_chunks),
        out_type=jax.ShapeDtypeStruct((NC, A_PAD, EMB), jnp.float32),
        mesh=mesh,
        scratch_types=[
            pltpu.VMEM((CHUNK,), jnp.int32),
            pltpu.VMEM((CHUNK,), jnp.int32),
            pltpu.VMEM((CHUNK,), jnp.int32),
            pltpu.VMEM((CHUNK,), jnp.int32),
            pltpu.VMEM((CHUNK // 2, EMB), jnp.uint32),
            pltpu.VMEM((CHUNK // 2, EMB), jnp.uint32),
            pltpu.VMEM((CHUNK, EMB), jnp.float32),
            pltpu.VMEM((CHUNK, EMB), jnp.float32),
            pltpu.VMEM((ZROWS, EMB), jnp.float32),
            pltpu.VMEM_SHARED((A_PAD, EMB), jnp.float32),
            pltpu.SemaphoreType.DMA,
            pltpu.SemaphoreType.DMA,
            pltpu.SemaphoreType.DMA,
            pltpu.SemaphoreType.DMA,
            pltpu.SemaphoreType.DMA,
            pltpu.SemaphoreType.DMA,
        ],
        compiler_params=pltpu.CompilerParams(needs_layout_passes=False),
    )
    return k(x, idx_atom)


def _mlp_body(scale_ref, pa_ref, pb_ref, w10_ref, w20_ref, w11_ref, w21_ref,
              out_ref):
    x = (pa_ref[0] + pa_ref[1] + pb_ref[0] + pb_ref[1]) * scale_ref[0]
    for wa_ref, wb_ref in ((w10_ref, w20_ref), (w11_ref, w21_ref)):
        y = _scaled_silu(jnp.dot(x, wa_ref[...], preferred_element_type=jnp.float32))
        y = _scaled_silu(jnp.dot(y, wb_ref[...], preferred_element_type=jnp.float32))
        x = (x + y) * _INV_SQRT2
    out_ref[...] = x


def _residual_mlp(pa, pb, scale, W1_0, W2_0, W1_1, W2_1, block=2000):
    grid = (N_ATOMS // block,)
    wspec = pl.BlockSpec((EMB, EMB), lambda i: (0, 0))
    pspec = pl.BlockSpec((NC, block, EMB), lambda i: (0, i, 0))
    return pl.pallas_call(
        _mlp_body,
        grid=grid,
        in_specs=[
            pl.BlockSpec(memory_space=pltpu.SMEM),
            pspec, pspec,
            wspec, wspec, wspec, wspec,
        ],
        out_specs=pl.BlockSpec((block, EMB), lambda i: (i, 0)),
        out_shape=jax.ShapeDtypeStruct((N_ATOMS, EMB), jnp.float32),
    )(scale.reshape((1,)), pa, pb, W1_0, W2_0, W1_1, W2_1)


def kernel(h, m, basis_rad, idx_atom, W_rbf, W1_0, W2_0, W1_1, W2_1, scale):
    idx32 = idx_atom.astype(jnp.int32)
    xa = _edge_messages(basis_rad, m, W_rbf, 0, NBLK_A)
    pa = _scatter_partials(xa, idx32, 0, CHUNKS_A)
    xb = _edge_messages(basis_rad, m, W_rbf, NBLK_A, NBLK_B)
    pb = _scatter_partials(xb, idx32, E_PART_A, CHUNKS_B)
    return _residual_mlp(pa, pb, scale, W1_0, W2_0, W1_1, W2_1)
